# Initial kernel scaffold; baseline (speedup 1.0000x reference)
import jax
import jax.numpy as jnp
from jax.experimental import pallas as pl


def kernel(x, edge_index, edge_weight, W1, b1, W2, b2):
    raise NotImplementedError("write your pallas kernel here")


# speedup vs baseline: 24.4656x; 24.4656x over previous
import functools

import jax
import jax.numpy as jnp
from jax import lax
from jax.experimental import pallas as pl
from jax.experimental.pallas import tpu as pltpu
from jax.experimental.pallas import tpu_sc as plsc

N = 10000
D_IN = 128
D_HID = 64
NC = 2
NS = 16
L = 16
NP = 10240
ROWS_PER_TILE = NP // NS
CH = 128

_mesh = plsc.VectorSubcoreMesh(
    core_axis_name="c", subcore_axis_name="s", num_cores=NC, num_subcores=NS)
_sc_params = pltpu.CompilerParams(
    needs_layout_passes=False, use_tc_tiling_on_sc=False)


def _num_chunks(E):
  return -(-E // (NC * NS * CH))


def _make_scalar_seg_sum(cpt, with_table):
  scratch = [
      pltpu.VMEM((cpt, CH), jnp.int32),
      pltpu.VMEM((cpt * CH,), jnp.float32),
      pltpu.VMEM((NP,), jnp.float32),
      pltpu.VMEM((NS, ROWS_PER_TILE), jnp.float32),
      pltpu.VMEM_SHARED((NS, NP), jnp.float32),
  ]
  if with_table:
    scratch = [
        pltpu.VMEM((cpt, CH), jnp.int32),
        pltpu.VMEM((NP,), jnp.float32),
    ] + scratch

  def body(*refs):
    if with_table:
      (row_h, col_h, ew_h, z_h, out_h,
       row_v, z_v, col_v, ew_v, acc_v, m_v, stage_s) = refs
    else:
      (col_h, ew_h, out_h, col_v, ew_v, acc_v, m_v, stage_s) = refs
    c = lax.axis_index("c")
    s = lax.axis_index("s")
    wid = c * NS + s

    zeros = jnp.zeros((L,), jnp.float32)

    def zero_body(i, _):
      acc_v[pl.ds(i * L, L)] = zeros
      return 0
    lax.fori_loop(0, NP // L, zero_body, 0)

    pltpu.sync_copy(col_h.at[wid], col_v)
    pltpu.sync_copy(ew_h.at[wid], ew_v)
    if with_table:
      pltpu.sync_copy(row_h.at[wid], row_v)
      pltpu.sync_copy(z_h, z_v)

    def edge_body(g, _):
      j = g // (CH // L)
      o = (g % (CH // L)) * L
      c16 = col_v[j, pl.ds(o, L)]
      w16 = ew_v[pl.ds(g * L, L)]
      if with_table:
        r16 = row_v[j, pl.ds(o, L)]
        w16 = w16 * plsc.load_gather(z_v, [r16])
      plsc.addupdate_scatter(acc_v, [c16], w16)
      return 0
    lax.fori_loop(0, cpt * (CH // L), edge_body, 0)

    pltpu.sync_copy(acc_v, stage_s.at[s])
    plsc.subcore_barrier()
    pltpu.sync_copy(stage_s.at[:, pl.ds(s * ROWS_PER_TILE, ROWS_PER_TILE)],
                    m_v)

    def merge_body(p, _):
      v = m_v[0, pl.ds(p * L, L)]
      for t in range(1, NS):
        v = v + m_v[t, pl.ds(p * L, L)]
      acc_v[pl.ds(p * L, L)] = v
      return 0
    lax.fori_loop(0, ROWS_PER_TILE // L, merge_body, 0)
    pltpu.sync_copy(acc_v.at[pl.ds(0, ROWS_PER_TILE)],
                    out_h.at[c, pl.ds(s * ROWS_PER_TILE, ROWS_PER_TILE)])

  return functools.partial(
      pl.kernel, body, mesh=_mesh,
      out_type=jax.ShapeDtypeStruct((NC, NP), jnp.float32),
      compiler_params=_sc_params,
      scratch_types=scratch)


def _make_agg64(cpt):
  def body(row_h, col_h, ew_h, y_h, out_h,
           row_v, col_v, ew_v, r_v, acc_s, sem):
    c = lax.axis_index("c")
    s = lax.axis_index("s")
    wid = c * NS + s

    zeros = jnp.zeros((L,), jnp.float32)

    def zero_body(i, _):
      for f in range(D_HID // L):
        r_v[i, pl.ds(f * L, L)] = zeros
      return 0
    lax.fori_loop(0, CH, zero_body, 0)
    for k in range(ROWS_PER_TILE // CH):
      pltpu.sync_copy(r_v, acc_s.at[pl.ds(s * ROWS_PER_TILE + k * CH, CH)])
    plsc.subcore_barrier()

    pltpu.sync_copy(row_h.at[wid], row_v)
    pltpu.sync_copy(col_h.at[wid], col_v)
    pltpu.sync_copy(ew_h.at[wid], ew_v)

    def chunk_body(j, _):
      pltpu.async_copy(y_h.at[row_v.at[j]], r_v, sem).wait()

      def group_body(g, _):
        base = j * CH + g * L
        for e in range(L):
          w = plsc.load_gather(ew_v, [jnp.full((L,), base + e, jnp.int32)])
          r = g * L + e
          for f in range(D_HID // L):
            r_v[r, pl.ds(f * L, L)] = r_v[r, pl.ds(f * L, L)] * w
        return 0
      lax.fori_loop(0, CH // L, group_body, 0)
      pltpu.sync_copy(r_v, acc_s.at[col_v.at[j]], add=True)
      return 0
    lax.fori_loop(0, cpt, chunk_body, 0)

    plsc.subcore_barrier()
    pltpu.sync_copy(
        acc_s.at[pl.ds(s * ROWS_PER_TILE, ROWS_PER_TILE)],
        out_h.at[c, pl.ds(s * ROWS_PER_TILE, ROWS_PER_TILE)])

  return functools.partial(
      pl.kernel, body, mesh=_mesh,
      out_type=jax.ShapeDtypeStruct((NC, NP, D_HID), jnp.float32),
      compiler_params=_sc_params,
      scratch_types=[
          pltpu.VMEM((cpt, CH), jnp.int32),
          pltpu.VMEM((cpt, CH), jnp.int32),
          pltpu.VMEM((cpt * CH,), jnp.float32),
          pltpu.VMEM((CH, D_HID), jnp.float32),
          pltpu.VMEM_SHARED((NP, D_HID), jnp.float32),
          pltpu.SemaphoreType.DMA,
      ])


_BLK = 1000


def _dense1_body(x_ref, w1_ref, deg_ref, y_ref, dinv_ref):
  deg = deg_ref[:, 0] + deg_ref[:, 1] + 1.0
  dinv = jnp.where(deg > 0, lax.rsqrt(deg), 0.0)
  xw = jnp.dot(x_ref[...], w1_ref[...], preferred_element_type=jnp.float32)
  y_ref[...] = xw * dinv[:, None]
  dinv_ref[...] = dinv[:, None]


def _dense1(x, W1, deg2t):
  grid = (N // _BLK,)
  return pl.pallas_call(
      _dense1_body,
      grid=grid,
      in_specs=[
          pl.BlockSpec((_BLK, D_IN), lambda i: (i, 0)),
          pl.BlockSpec((D_IN, D_HID), lambda i: (0, 0)),
          pl.BlockSpec((_BLK, NC), lambda i: (i, 0)),
      ],
      out_specs=[
          pl.BlockSpec((_BLK, D_HID), lambda i: (i, 0)),
          pl.BlockSpec((_BLK, 1), lambda i: (i, 0)),
      ],
      out_shape=[
          jax.ShapeDtypeStruct((N, D_HID), jnp.float32),
          jax.ShapeDtypeStruct((N, 1), jnp.float32),
      ],
  )(x, W1, deg2t)


def _dense2_body(acc1_ref, y_ref, dinv_ref, b1_ref, w2_ref, z_ref):
  a = acc1_ref[0] + acc1_ref[1] + y_ref[...]
  pre = dinv_ref[...] * a + b1_ref[...]
  h = jnp.where(pre > 0, pre, jnp.exp(pre) - 1.0)
  z_ref[...] = jnp.dot(
      h, w2_ref[...], preferred_element_type=jnp.float32) * dinv_ref[...]


def _dense2(acc1, y, dinv, b1, W2):
  grid = (N // _BLK,)
  return pl.pallas_call(
      _dense2_body,
      grid=grid,
      in_specs=[
          pl.BlockSpec((NC, _BLK, D_HID), lambda i: (0, i, 0)),
          pl.BlockSpec((_BLK, D_HID), lambda i: (i, 0)),
          pl.BlockSpec((_BLK, 1), lambda i: (i, 0)),
          pl.BlockSpec((1, D_HID), lambda i: (0, 0)),
          pl.BlockSpec((D_HID, 1), lambda i: (0, 0)),
      ],
      out_specs=pl.BlockSpec((_BLK, 1), lambda i: (i, 0)),
      out_shape=jax.ShapeDtypeStruct((N, 1), jnp.float32),
  )(acc1, y, dinv, b1, W2)


def _final_body(acc2_ref, z_ref, dinv_ref, b2_ref, out_ref):
  t = acc2_ref[:, 0:1] + acc2_ref[:, 1:2]
  pre = dinv_ref[...] * (t + z_ref[...]) + b2_ref[...]
  out_ref[...] = 1.0 / (1.0 + jnp.exp(-pre))


def _final(acc2t, z, dinv, b2):
  grid = (N // _BLK,)
  return pl.pallas_call(
      _final_body,
      grid=grid,
      in_specs=[
          pl.BlockSpec((_BLK, NC), lambda i: (i, 0)),
          pl.BlockSpec((_BLK, 1), lambda i: (i, 0)),
          pl.BlockSpec((_BLK, 1), lambda i: (i, 0)),
          pl.BlockSpec((1, 1), lambda i: (0, 0)),
      ],
      out_specs=pl.BlockSpec((_BLK, 1), lambda i: (i, 0)),
      out_shape=jax.ShapeDtypeStruct((N, 1), jnp.float32),
  )(acc2t, z, dinv, b2)


def kernel(x, edge_index, edge_weight, W1, b1, W2, b2):
  E = edge_weight.shape[0]
  cpt = _num_chunks(E)
  ep = NC * NS * cpt * CH

  row = edge_index[0]
  col = edge_index[1]
  row_p = jnp.pad(row, (0, ep - E)).reshape(NC * NS, cpt, CH)
  col_p = jnp.pad(col, (0, ep - E)).reshape(NC * NS, cpt, CH)
  ew_p = jnp.pad(edge_weight, (0, ep - E)).reshape(NC * NS, cpt * CH)

  deg2 = _make_scalar_seg_sum(cpt, with_table=False)()(col_p, ew_p)
  y, dinv = _dense1(x, W1, deg2[:, :N].T)
  acc1 = _make_agg64(cpt)()(row_p, col_p, ew_p, y)
  z = _dense2(acc1[:, :N], y, dinv, b1.reshape(1, D_HID), W2)
  z_pad = jnp.pad(z[:, 0], (0, NP - N))
  acc2 = _make_scalar_seg_sum(cpt, with_table=True)()(
      row_p, col_p, ew_p, z_pad)
  return _final(acc2[:, :N].T, z, dinv, b2.reshape(1, 1))
